# baseline (speedup 1.0000x reference)
import jax
import jax.numpy as jnp
from jax.experimental import pallas as pl

_N = 10000
_E = 160000
_H = 256
_ED = 768
_NL = 6
_LANE = 128


def _mm_kernel(x_ref, w_ref, o_ref):
    o_ref[...] = jnp.dot(x_ref[...], w_ref[...],
                         preferred_element_type=jnp.float32)


def _mm(x, w, bm):
    m, k = x.shape
    n = w.shape[1]
    return pl.pallas_call(
        _mm_kernel,
        grid=(m // bm,),
        in_specs=[pl.BlockSpec((bm, k), lambda i: (i, 0)),
                  pl.BlockSpec((k, n), lambda i: (0, 0))],
        out_specs=pl.BlockSpec((bm, n), lambda i: (i, 0)),
        out_shape=jax.ShapeDtypeStruct((m, n), jnp.float32),
    )(x, w)


def _node_kernel(h_ref, w_ref, a_ref, hp_ref, al_ref):
    hp = jnp.dot(h_ref[...], w_ref[...], preferred_element_type=jnp.float32)
    hp_ref[...] = hp
    al_ref[...] = jnp.dot(hp, a_ref[...], preferred_element_type=jnp.float32)


def _node(h, w, a2, bm):
    m = h.shape[0]
    return pl.pallas_call(
        _node_kernel,
        grid=(m // bm,),
        in_specs=[pl.BlockSpec((bm, _H), lambda i: (i, 0)),
                  pl.BlockSpec((_H, _H), lambda i: (0, 0)),
                  pl.BlockSpec((_H, _LANE), lambda i: (0, 0))],
        out_specs=[pl.BlockSpec((bm, _H), lambda i: (i, 0)),
                   pl.BlockSpec((bm, _LANE), lambda i: (i, 0))],
        out_shape=[jax.ShapeDtypeStruct((m, _H), jnp.float32),
                   jax.ShapeDtypeStruct((m, _LANE), jnp.float32)],
    )(h, w, a2)


def _final_kernel(h_ref, w_ref, o_ref):
    o_ref[...] = jnp.dot(jnp.maximum(h_ref[...], 0.0), w_ref[...],
                         preferred_element_type=jnp.float32)


def _final(h, w, bm):
    m = h.shape[0]
    n = w.shape[1]
    return pl.pallas_call(
        _final_kernel,
        grid=(m // bm,),
        in_specs=[pl.BlockSpec((bm, _H), lambda i: (i, 0)),
                  pl.BlockSpec((_H, n), lambda i: (0, 0))],
        out_specs=pl.BlockSpec((bm, n), lambda i: (i, 0)),
        out_shape=jax.ShapeDtypeStruct((m, n), jnp.float32),
    )(h, w)


def kernel(x, edge_index, edge_attr, W1, W2, Wc, att_src, att_dst, We,
           att_edge, conv_bias, W3):
    order = jnp.argsort(edge_index[1])
    src = edge_index[0][order]
    dst = edge_index[1][order]

    V = jnp.einsum('lij,lj->il', We, att_edge)
    Vpad = jnp.zeros((_ED, _LANE), jnp.float32).at[:, :_NL].set(V)
    AE = _mm(edge_attr, Vpad, bm=1000)[order, :_NL].T

    W12 = _mm(W1, W2, bm=_H)
    h = _mm(x, W12, bm=1000)

    for i in range(_NL):
        a2 = jnp.zeros((_H, _LANE), jnp.float32)
        a2 = a2.at[:, 0].set(att_src[i]).at[:, 1].set(att_dst[i])
        hp, al = _node(h, Wc[i], a2, bm=1000)
        asrc = al[:, 0]
        adst = al[:, 1]
        alpha = asrc[src] + adst[dst] + AE[i]
        alpha = jnp.where(alpha > 0, alpha, 0.2 * alpha)
        amax = jax.ops.segment_max(alpha, dst, num_segments=_N,
                                   indices_are_sorted=True)
        ex = jnp.exp(alpha - amax[dst])
        denom = jax.ops.segment_sum(ex, dst, num_segments=_N,
                                    indices_are_sorted=True)
        coef = ex / (denom[dst] + 1e-16)
        out = jax.ops.segment_sum(hp[src] * coef[:, None], dst,
                                  num_segments=_N, indices_are_sorted=True)
        h = out + conv_bias[i]

    W3c = W3[:_H] + W3[_H:]
    return _final(h, W3c, bm=1000)


# speedup vs baseline: 9.6821x; 9.6821x over previous
import jax
import jax.numpy as jnp
from jax.experimental import pallas as pl

_N = 10000
_E = 160000
_H = 256
_ED = 768
_NL = 6
_LANE = 128


def _mm_kernel(x_ref, w_ref, o_ref):
    o_ref[...] = jnp.dot(x_ref[...], w_ref[...],
                         preferred_element_type=jnp.float32)


def _mm(x, w, bm):
    m, k = x.shape
    n = w.shape[1]
    return pl.pallas_call(
        _mm_kernel,
        grid=(m // bm,),
        in_specs=[pl.BlockSpec((bm, k), lambda i: (i, 0)),
                  pl.BlockSpec((k, n), lambda i: (0, 0))],
        out_specs=pl.BlockSpec((bm, n), lambda i: (i, 0)),
        out_shape=jax.ShapeDtypeStruct((m, n), jnp.float32),
    )(x, w)


def _node_kernel(h_ref, w_ref, a_ref, hp_ref, al_ref):
    hp = jnp.dot(h_ref[...], w_ref[...], preferred_element_type=jnp.float32)
    hp_ref[...] = hp
    al_ref[...] = jnp.dot(hp, a_ref[...], preferred_element_type=jnp.float32)


def _node(h, w, a2, bm):
    m = h.shape[0]
    return pl.pallas_call(
        _node_kernel,
        grid=(m // bm,),
        in_specs=[pl.BlockSpec((bm, _H), lambda i: (i, 0)),
                  pl.BlockSpec((_H, _H), lambda i: (0, 0)),
                  pl.BlockSpec((_H, _LANE), lambda i: (0, 0))],
        out_specs=[pl.BlockSpec((bm, _H), lambda i: (i, 0)),
                   pl.BlockSpec((bm, _LANE), lambda i: (i, 0))],
        out_shape=[jax.ShapeDtypeStruct((m, _H), jnp.float32),
                   jax.ShapeDtypeStruct((m, _LANE), jnp.float32)],
    )(h, w, a2)


def _final_kernel(h_ref, w_ref, o_ref):
    o_ref[...] = jnp.dot(jnp.maximum(h_ref[...], 0.0), w_ref[...],
                         preferred_element_type=jnp.float32)


def _final(h, w, bm):
    m = h.shape[0]
    n = w.shape[1]
    return pl.pallas_call(
        _final_kernel,
        grid=(m // bm,),
        in_specs=[pl.BlockSpec((bm, _H), lambda i: (i, 0)),
                  pl.BlockSpec((_H, n), lambda i: (0, 0))],
        out_specs=pl.BlockSpec((bm, n), lambda i: (i, 0)),
        out_shape=jax.ShapeDtypeStruct((m, n), jnp.float32),
    )(h, w)


def kernel(x, edge_index, edge_attr, W1, W2, Wc, att_src, att_dst, We,
           att_edge, conv_bias, W3):
    order = jnp.argsort(edge_index[1])
    src = edge_index[0][order]
    dst = edge_index[1][order]

    V = jnp.einsum('lij,lj->il', We, att_edge)
    Vpad = jnp.zeros((_ED, _LANE), jnp.float32).at[:, :_NL].set(V)
    AE = _mm(edge_attr, Vpad, bm=1000)[order]

    W12 = _mm(W1, W2, bm=_H)
    h = _mm(x, W12, bm=1000)

    for i in range(_NL):
        a2 = jnp.zeros((_H, _LANE), jnp.float32)
        a2 = a2.at[:, 0].set(att_src[i]).at[:, 1].set(att_dst[i])
        hp, al = _node(h, Wc[i], a2, bm=1000)
        alpha = al[src, 0] + al[dst, 1] + AE[:, i]
        alpha = jnp.where(alpha > 0, alpha, 0.2 * alpha)
        amax = jax.ops.segment_max(alpha, dst, num_segments=_N,
                                   indices_are_sorted=True)
        ex = jnp.exp(alpha - amax[dst])
        denom = jax.ops.segment_sum(ex, dst, num_segments=_N,
                                    indices_are_sorted=True)
        coef = ex / (denom[dst] + 1e-16)
        out = jax.ops.segment_sum(hp[src] * coef[:, None], dst,
                                  num_segments=_N, indices_are_sorted=True)
        h = out + conv_bias[i]

    W3c = W3[:_H] + W3[_H:]
    return _final(h, W3c, bm=1000)
